# baseline (speedup 1.0000x reference)
import functools

import jax
import jax.numpy as jnp
import numpy as np
from jax.experimental import pallas as pl
from jax.experimental.pallas import tpu as pltpu

_B, _T, _H, _D, _E, _K = 1, 2048, 16, 64, 8, 2
_C = _H * _D
_SCALE = 1.0 / np.sqrt(_D)
_BQ = 512
_BM = 256


def _route_project(xb, xb_bf, wr_ref, wef_ref, be_ref, l_ref):
    logits = jnp.dot(xb, wr_ref[...], preferred_element_type=jnp.float32)
    l_ref[0] = logits
    m = jnp.max(logits, axis=-1, keepdims=True)
    ex = jnp.exp(logits - m)
    p = ex / jnp.sum(ex, axis=-1, keepdims=True)
    p1 = jnp.max(p, axis=-1, keepdims=True)
    i1 = jnp.argmax(p, axis=-1)
    eidx = jax.lax.broadcasted_iota(jnp.int32, (_T, _E), 1)
    not_first = eidx != i1[:, None]
    p2 = jnp.max(jnp.where(not_first, p, -1.0), axis=-1, keepdims=True)
    sel = p >= p2
    wd = jnp.where(sel, p, 0.0) / (p1 + p2)
    y = jnp.dot(xb_bf, wef_ref[...], preferred_element_type=jnp.float32)
    ci = jax.lax.broadcasted_iota(jnp.int32, (_E, _E * _D), 1)
    ei = jax.lax.broadcasted_iota(jnp.int32, (_E, _E * _D), 0)
    s_mat = (ci // _D == ei).astype(jnp.float32)
    wdexp = jnp.dot(wd, s_mat, preferred_element_type=jnp.float32)
    z = (wdexp * (y + be_ref[...])).astype(jnp.bfloat16)
    fi = jax.lax.broadcasted_iota(jnp.int32, (_E * _D, _D), 0)
    fj = jax.lax.broadcasted_iota(jnp.int32, (_E * _D, _D), 1)
    f_mat = (fi % _D == fj).astype(jnp.bfloat16)
    return jnp.dot(z, f_mat, preferred_element_type=jnp.float32)


def _proj_kernel(xh_ref,
                 wrq_ref, wrk_ref, wrv_ref,
                 weq_ref, wek_ref, wev_ref,
                 beq_ref, bek_ref, bev_ref,
                 q_ref, k_ref, v_ref, ql_ref, kl_ref, vl_ref):
    xb = xh_ref[0]
    xb_bf = xb.astype(jnp.bfloat16)
    q_ref[0] = _route_project(xb, xb_bf, wrq_ref, weq_ref, beq_ref,
                              ql_ref).astype(jnp.bfloat16)
    k_ref[0] = _route_project(xb, xb_bf, wrk_ref, wek_ref, bek_ref,
                              kl_ref).astype(jnp.bfloat16)
    v = _route_project(xb, xb_bf, wrv_ref, wev_ref, bev_ref, vl_ref)
    pad = (jax.lax.broadcasted_iota(jnp.int32, (_T, _D), 1) == 0)
    v_ref[0] = jnp.concatenate(
        [v, pad.astype(jnp.float32)], axis=1).astype(jnp.bfloat16)


def _attn_kernel(q_ref, k_ref, vaug_ref, o_ref):
    q = q_ref[0]
    k = k_ref[0]
    s = jax.lax.dot_general(q, k, (((1,), (1,)), ((), ())),
                            preferred_element_type=jnp.float32) * _SCALE
    ti = jax.lax.broadcasted_iota(jnp.int32, (_T, _T), 0)
    si = jax.lax.broadcasted_iota(jnp.int32, (_T, _T), 1)
    s = jnp.where(si <= ti, s, -jnp.inf)
    m = jnp.max(s, axis=-1, keepdims=True)
    p = jnp.exp(s - m).astype(jnp.bfloat16)
    pv = jnp.dot(p, vaug_ref[0], preferred_element_type=jnp.float32)
    o_ref[0] = (pv[:, :_D] / pv[:, _D:_D + 1]).astype(jnp.bfloat16)


def _oproj_kernel(o_ref, wo_ref, out_ref):
    out_ref[...] = jax.lax.dot_general(
        o_ref[...], wo_ref[...], (((1,), (1,)), ((), ())),
        preferred_element_type=jnp.float32)


@functools.partial(jax.jit, static_argnames=("interpret",))
def kernel(x, Wr_q, Wr_k, Wr_v, We_q, be_q, We_k, be_k, We_v, be_v, Wo,
           interpret=False):
    xh = x.reshape(_T, _H, _D).transpose(1, 0, 2)
    bf = jnp.bfloat16
    wef_q = We_q.transpose(1, 0, 2).reshape(_D, _E * _D).astype(bf)
    wef_k = We_k.transpose(1, 0, 2).reshape(_D, _E * _D).astype(bf)
    wef_v = We_v.transpose(1, 0, 2).reshape(_D, _E * _D).astype(bf)

    full = lambda shape: pl.BlockSpec(shape, lambda h: (0,) * len(shape))
    head_blk = pl.BlockSpec((1, _T, _D), lambda h: (h, 0, 0))
    logit_blk = pl.BlockSpec((1, _T, _E), lambda h: (h, 0, 0))

    q, k, v, ql, kl, vl = pl.pallas_call(
        _proj_kernel,
        grid=(_H,),
        in_specs=[
            head_blk,
            full((_D, _E)), full((_D, _E)), full((_D, _E)),
            full((_D, _E * _D)), full((_D, _E * _D)), full((_D, _E * _D)),
            full((1, _E * _D)), full((1, _E * _D)), full((1, _E * _D)),
        ],
        out_specs=[head_blk, head_blk,
                   pl.BlockSpec((1, _T, 2 * _D), lambda h: (h, 0, 0)),
                   logit_blk, logit_blk, logit_blk],
        out_shape=[
            jax.ShapeDtypeStruct((_H, _T, _D), bf),
            jax.ShapeDtypeStruct((_H, _T, _D), bf),
            jax.ShapeDtypeStruct((_H, _T, 2 * _D), bf),
            jax.ShapeDtypeStruct((_H, _T, _E), jnp.float32),
            jax.ShapeDtypeStruct((_H, _T, _E), jnp.float32),
            jax.ShapeDtypeStruct((_H, _T, _E), jnp.float32),
        ],
        interpret=interpret,
    )(xh, Wr_q, Wr_k, Wr_v, wef_q, wef_k, wef_v,
      be_q.reshape(1, _E * _D), be_k.reshape(1, _E * _D),
      be_v.reshape(1, _E * _D))

    o = pl.pallas_call(
        _attn_kernel,
        grid=(_H,),
        in_specs=[
            pl.BlockSpec((1, _T, _D), lambda h: (h, 0, 0)),
            pl.BlockSpec((1, _T, _D), lambda h: (h, 0, 0)),
            pl.BlockSpec((1, _T, 2 * _D), lambda h: (h, 0, 0)),
        ],
        out_specs=pl.BlockSpec((1, _T, _D), lambda h: (h, 0, 0)),
        out_shape=jax.ShapeDtypeStruct((_H, _T, _D), bf),
        interpret=interpret,
    )(q, k, v)

    o_flat = o.transpose(1, 0, 2).reshape(_T, _C)
    out = pl.pallas_call(
        _oproj_kernel,
        grid=(_T // _BM,),
        in_specs=[
            pl.BlockSpec((_BM, _C), lambda i: (i, 0)),
            pl.BlockSpec((_C, _C), lambda i: (0, 0)),
        ],
        out_specs=pl.BlockSpec((_BM, _C), lambda i: (i, 0)),
        out_shape=jax.ShapeDtypeStruct((_T, _C), jnp.float32),
        interpret=interpret,
    )(o_flat, Wo.astype(bf))

    tr = lambda a: a.transpose(1, 0, 2)[None]
    return out[None], tr(ql), tr(kl), tr(vl)


# speedup vs baseline: 1.0759x; 1.0759x over previous
import functools

import jax
import jax.numpy as jnp
import numpy as np
from jax.experimental import pallas as pl
from jax.experimental.pallas import tpu as pltpu

_B, _T, _H, _D, _E, _K = 1, 2048, 16, 64, 8, 2
_C = _H * _D
_SCALE = 1.0 / np.sqrt(_D)
_NC = 4
_BC = _T // _NC
_BM = 256


def _route_project(xb, xb_bf, wr_ref, wef_ref, be_ref, l_ref):
    logits = jnp.dot(xb, wr_ref[...], preferred_element_type=jnp.float32)
    l_ref[0] = logits
    m = jnp.max(logits, axis=-1, keepdims=True)
    ex = jnp.exp(logits - m)
    p = ex / jnp.sum(ex, axis=-1, keepdims=True)
    p1 = jnp.max(p, axis=-1, keepdims=True)
    i1 = jnp.argmax(p, axis=-1)
    eidx = jax.lax.broadcasted_iota(jnp.int32, (_T, _E), 1)
    not_first = eidx != i1[:, None]
    p2 = jnp.max(jnp.where(not_first, p, -1.0), axis=-1, keepdims=True)
    sel = p >= p2
    wd = jnp.where(sel, p, 0.0) / (p1 + p2)
    y = jnp.dot(xb_bf, wef_ref[...], preferred_element_type=jnp.float32)
    ci = jax.lax.broadcasted_iota(jnp.int32, (_E, _E * _D), 1)
    ei = jax.lax.broadcasted_iota(jnp.int32, (_E, _E * _D), 0)
    s_mat = (ci // _D == ei).astype(jnp.float32)
    wdexp = jnp.dot(wd, s_mat, preferred_element_type=jnp.float32)
    z = (wdexp * (y + be_ref[...])).astype(jnp.bfloat16)
    fi = jax.lax.broadcasted_iota(jnp.int32, (_E * _D, _D), 0)
    fj = jax.lax.broadcasted_iota(jnp.int32, (_E * _D, _D), 1)
    f_mat = (fi % _D == fj).astype(jnp.bfloat16)
    return jnp.dot(z, f_mat, preferred_element_type=jnp.float32)


def _mega_kernel(xh_ref,
                 wrq_ref, wrk_ref, wrv_ref,
                 weq_ref, wek_ref, wev_ref,
                 beq_ref, bek_ref, bev_ref,
                 ql_ref, kl_ref, vl_ref, o_ref):
    xb = xh_ref[0]
    xb_bf = xb.astype(jnp.bfloat16)
    q = _route_project(xb, xb_bf, wrq_ref, weq_ref, beq_ref,
                       ql_ref).astype(jnp.bfloat16)
    k = _route_project(xb, xb_bf, wrk_ref, wek_ref, bek_ref,
                       kl_ref).astype(jnp.bfloat16)
    v = _route_project(xb, xb_bf, wrv_ref, wev_ref, bev_ref, vl_ref)
    ones_col = (jax.lax.broadcasted_iota(jnp.int32, (_T, _D), 1) == 0)
    vaug = jnp.concatenate(
        [v, ones_col.astype(jnp.float32)], axis=1).astype(jnp.bfloat16)

    o_parts = []
    for c in range(_NC):
        ln = (c + 1) * _BC
        qs = q[c * _BC:(c + 1) * _BC]
        kc = k[:ln]
        s = jax.lax.dot_general(qs, kc, (((1,), (1,)), ((), ())),
                                preferred_element_type=jnp.float32) * _SCALE
        ti = jax.lax.broadcasted_iota(jnp.int32, (_BC, ln), 0) + c * _BC
        si = jax.lax.broadcasted_iota(jnp.int32, (_BC, ln), 1)
        s = jnp.where(si <= ti, s, -jnp.inf)
        m = jnp.max(s, axis=-1, keepdims=True)
        p = jnp.exp(s - m).astype(jnp.bfloat16)
        pv = jnp.dot(p, vaug[:ln], preferred_element_type=jnp.float32)
        o_parts.append(pv[:, :_D] / pv[:, _D:_D + 1])
    o = jnp.concatenate(o_parts, axis=0)
    o_ref[0] = o.astype(jnp.bfloat16)


def _oproj_kernel(o_ref, woh_ref, out_ref):
    h = pl.program_id(1)

    @pl.when(h == 0)
    def _():
        out_ref[...] = jnp.zeros_like(out_ref)

    out_ref[...] += jnp.dot(o_ref[0], woh_ref[h],
                            preferred_element_type=jnp.float32)


@functools.partial(jax.jit, static_argnames=("interpret",))
def kernel(x, Wr_q, Wr_k, Wr_v, We_q, be_q, We_k, be_k, We_v, be_v, Wo,
           interpret=False):
    xh = x.reshape(_T, _H, _D).transpose(1, 0, 2)
    bf = jnp.bfloat16
    wef_q = We_q.transpose(1, 0, 2).reshape(_D, _E * _D).astype(bf)
    wef_k = We_k.transpose(1, 0, 2).reshape(_D, _E * _D).astype(bf)
    wef_v = We_v.transpose(1, 0, 2).reshape(_D, _E * _D).astype(bf)
    woh = Wo.T.reshape(_H, _D, _C).astype(bf)

    full = lambda shape: pl.BlockSpec(shape, lambda h: (0,) * len(shape))
    head_blk = pl.BlockSpec((1, _T, _D), lambda h: (h, 0, 0))
    logit_blk = pl.BlockSpec((1, _T, _E), lambda h: (h, 0, 0))

    ql, kl, vl, o = pl.pallas_call(
        _mega_kernel,
        grid=(_H,),
        in_specs=[
            head_blk,
            full((_D, _E)), full((_D, _E)), full((_D, _E)),
            full((_D, _E * _D)), full((_D, _E * _D)), full((_D, _E * _D)),
            full((1, _E * _D)), full((1, _E * _D)), full((1, _E * _D)),
        ],
        out_specs=[logit_blk, logit_blk, logit_blk, head_blk],
        out_shape=[
            jax.ShapeDtypeStruct((_H, _T, _E), jnp.float32),
            jax.ShapeDtypeStruct((_H, _T, _E), jnp.float32),
            jax.ShapeDtypeStruct((_H, _T, _E), jnp.float32),
            jax.ShapeDtypeStruct((_H, _T, _D), bf),
        ],
        interpret=interpret,
    )(xh, Wr_q, Wr_k, Wr_v, wef_q, wef_k, wef_v,
      be_q.reshape(1, _E * _D), be_k.reshape(1, _E * _D),
      be_v.reshape(1, _E * _D))

    out = pl.pallas_call(
        _oproj_kernel,
        grid=(_T // _BM, _H),
        in_specs=[
            pl.BlockSpec((1, _BM, _D), lambda i, h: (h, i, 0)),
            pl.BlockSpec((_H, _D, _C), lambda i, h: (0, 0, 0)),
        ],
        out_specs=pl.BlockSpec((_BM, _C), lambda i, h: (i, 0)),
        out_shape=jax.ShapeDtypeStruct((_T, _C), jnp.float32),
        interpret=interpret,
    )(o, woh)

    tr = lambda a: a.transpose(1, 0, 2)[None]
    return out[None], tr(ql), tr(kl), tr(vl)
